# baseline (speedup 1.0000x reference)
import functools

import jax
import jax.numpy as jnp
from jax import lax
from jax.experimental import pallas as pl
from jax.experimental.pallas import tpu as pltpu
from jax.experimental.pallas import tpu_sc as plsc

FIELDS = 26
FEAT_PER_FIELD = 10000
TOTAL_FEAT = FIELDS * FEAT_PER_FIELD
KDIM = 8
BATCH = 4096
NWORKERS = 32
ROWS_PER_W = BATCH // NWORKERS

CFEAT = 200
NCH_A = TOTAL_FEAT // CFEAT
NVREG_A = CFEAT * KDIM // 16
REM_A = NCH_A % NWORKERS

ROWS_PER_CHUNK = 16
CHUNKS_B = ROWS_PER_W // ROWS_PER_CHUNK
LOOK = ROWS_PER_CHUNK * FIELDS
NGATH = LOOK // 104


FBLK = 16384
SUBB = 2048
GRID_A = -(-TOTAL_FEAT // FBLK)


def _svt_block(vt_ref, out_ref):
    for t in range(FBLK // SUBB):
        s = jnp.sum(vt_ref[:, :, t * SUBB:(t + 1) * SUBB], axis=0)
        s = s.reshape(KDIM, SUBB // 16, 16)
        s = jnp.transpose(s, (1, 2, 0))
        out_ref[t * (SUBB // 16):(t + 1) * (SUBB // 16), :] = (
            s.reshape(SUBB // 16, 128))


def _ffm_body(inp_hbm, sv_hbm, w_hbm, out_hbm,
              idx0, idx1, sub0, sub1, wi0, wi1, rows0, rows1, wr0, wr1, outv,
              sv0_, sv1_, sw0, sw1):
    cid = lax.axis_index("c")
    sid = lax.axis_index("s")
    wid = sid * 2 + cid
    wflat = wid * (ROWS_PER_W * FIELDS)
    io = lax.iota(jnp.int32, 16)
    permi = lax.rem(io + 8, 16)
    lane0 = io == 0
    jhalf = io // 8
    kidx = lax.rem(io, 8)
    idxb = (idx0, idx1)
    subb = (sub0, sub1)
    wib = (wi0, wi1)
    rowsb = (rows0, rows1)
    wrb = (wr0, wr1)
    semv = (sv0_, sv1_)
    semw = (sw0, sw1)

    def prep(cc, p):
        start = pl.multiple_of(wflat + cc * LOOK, 8)
        pltpu.sync_copy(inp_hbm.at[pl.ds(start, LOOK)], wib[p])
        for m in range(LOOK // 16):
            sl = pl.ds(m * 16, 16)
            offs = lax.rem(io + (m * 16) % FIELDS, FIELDS) * FEAT_PER_FIELD
            full = wib[p][sl] + offs
            wib[p][sl] = full
            idxb[p][sl] = lax.shift_right_logical(full, 4)
            subb[p][sl] = lax.bitwise_and(full, 15)
        for g in range(NGATH):
            isl = idxb[p].at[pl.ds(g * 104, 104)]
            wsl = wib[p].at[pl.ds(g * 104, 104)]
            pltpu.make_async_copy(
                sv_hbm.at[isl], rowsb[p].at[pl.ds(g * 104, 104)],
                semv[p]).start()
            pltpu.make_async_copy(
                w_hbm.at[wsl], wrb[p].at[pl.ds(g * 104, 104)],
                semw[p]).start()

    def compute(cc, p):
        for g in range(NGATH):
            isl = idxb[p].at[pl.ds(g * 104, 104)]
            wsl = wib[p].at[pl.ds(g * 104, 104)]
            pltpu.make_async_copy(
                sv_hbm.at[isl], rowsb[p].at[pl.ds(g * 104, 104)],
                semv[p]).wait()
            pltpu.make_async_copy(
                w_hbm.at[wsl], wrb[p].at[pl.ds(g * 104, 104)],
                semw[p]).wait()
        zero = jnp.zeros((16,), jnp.float32)
        for r in range(ROWS_PER_CHUNK):
            S16 = zero
            Q16 = zero
            for m in range(13):
                d0 = (r * FIELDS + 2 * m) + jhalf
                subg = plsc.load_gather(subb[p], [d0])
                gv = plsc.load_gather(rowsb[p], [d0, subg * 8 + kidx])
                S16 = S16 + gv
                Q16 = Q16 + gv * gv
            Su = S16 + S16.at[permi].get(mode="promise_in_bounds")
            s2dup = jnp.sum(Su * Su)
            qs = jnp.sum(Q16)
            w1i = r * FIELDS + io
            w2i = jnp.where(io < 10, r * FIELDS + 16 + io, 0)
            g1 = plsc.load_gather(wrb[p], [w1i])
            g2 = plsc.load_gather(wrb[p], [w2i], mask=io < 10)
            wsum = jnp.sum(g1 + jnp.where(io < 10, g2, 0.0))
            val = wsum + 0.25 * s2dup - 0.5 * qs
            vid = jnp.full((16,), cc * ROWS_PER_CHUNK + r, jnp.int32)
            plsc.store_scatter(outv, [vid],
                               jnp.full((16,), val, jnp.float32), mask=lane0)

    prep(0, 0)

    def loop_body(i, carry):
        for p in range(2):
            cc = i * 2 + p

            @pl.when(cc + 1 < CHUNKS_B)
            def _():
                prep(cc + 1, (p + 1) % 2)

            compute(cc, p)
        return carry

    lax.fori_loop(0, CHUNKS_B // 2, loop_body, 0)
    obase = pl.multiple_of(wid * ROWS_PER_W, 8)
    pltpu.sync_copy(outv, out_hbm.at[pl.ds(obase, ROWS_PER_W)])


@jax.jit
def _ffm(inputs, w, v):
    mesh = plsc.VectorSubcoreMesh(core_axis_name="c", subcore_axis_name="s")
    params = pltpu.CompilerParams(
        needs_layout_passes=False, use_tc_tiling_on_sc=False)

    vt = jnp.transpose(v, (1, 2, 0))
    sv = pl.pallas_call(
        _svt_block,
        grid=(GRID_A,),
        in_specs=[pl.BlockSpec((FIELDS, KDIM, FBLK), lambda i: (0, 0, i))],
        out_specs=pl.BlockSpec((FBLK // 16, 128), lambda i: (i, 0)),
        out_shape=jax.ShapeDtypeStruct((TOTAL_FEAT // 16, 128), jnp.float32),
    )(vt)

    ffm_fn = pl.kernel(
        _ffm_body,
        mesh=mesh,
        compiler_params=params,
        out_type=jax.ShapeDtypeStruct((BATCH,), jnp.float32),
        scratch_types=[
            pltpu.VMEM((LOOK,), jnp.int32),
            pltpu.VMEM((LOOK,), jnp.int32),
            pltpu.VMEM((LOOK,), jnp.int32),
            pltpu.VMEM((LOOK,), jnp.int32),
            pltpu.VMEM((LOOK,), jnp.int32),
            pltpu.VMEM((LOOK,), jnp.int32),
            pltpu.VMEM((LOOK, 128), jnp.float32),
            pltpu.VMEM((LOOK, 128), jnp.float32),
            pltpu.VMEM((LOOK,), jnp.float32),
            pltpu.VMEM((LOOK,), jnp.float32),
            pltpu.VMEM((ROWS_PER_W,), jnp.float32),
            pltpu.SemaphoreType.DMA,
            pltpu.SemaphoreType.DMA,
            pltpu.SemaphoreType.DMA,
            pltpu.SemaphoreType.DMA,
        ],
    )
    return ffm_fn(inputs.reshape(-1), sv, w.reshape(-1))


def kernel(inputs, w0, w, v):
    out = _ffm(inputs, w, v)
    return out.reshape(BATCH, 1) + w0


# speedup vs baseline: 1.0355x; 1.0355x over previous
import functools

import jax
import jax.numpy as jnp
from jax import lax
from jax.experimental import pallas as pl
from jax.experimental.pallas import tpu as pltpu
from jax.experimental.pallas import tpu_sc as plsc

FIELDS = 26
FEAT_PER_FIELD = 10000
TOTAL_FEAT = FIELDS * FEAT_PER_FIELD
KDIM = 8
BATCH = 4096
NWORKERS = 32
ROWS_PER_W = BATCH // NWORKERS

ROWS_PER_CHUNK = 16
CHUNKS_B = ROWS_PER_W // ROWS_PER_CHUNK
LOOK = ROWS_PER_CHUNK * FIELDS
NGATH = LOOK // 104


FBLK = 16384
GRID_A = -(-TOTAL_FEAT // FBLK)


def _svt_block(vt_ref, out_ref):
    s = jnp.sum(vt_ref[...], axis=0)
    s = s.reshape(KDIM, FBLK // 16, 16)
    s = jnp.transpose(s, (1, 2, 0))
    out_ref[...] = s.reshape(FBLK // 16, 128)


def _ffm_body(inp_hbm, sv_hbm, w_hbm, out_hbm,
              idx0, idx1, sub0, sub1, wi0, wi1, rows0, rows1, wr0, wr1, outv,
              sv0_, sv1_, sw0, sw1):
    cid = lax.axis_index("c")
    sid = lax.axis_index("s")
    wid = sid * 2 + cid
    wflat = wid * (ROWS_PER_W * FIELDS)
    io = lax.iota(jnp.int32, 16)
    permi = lax.rem(io + 8, 16)
    lane0 = io == 0
    jhalf = io // 8
    kidx = lax.rem(io, 8)
    idxb = (idx0, idx1)
    subb = (sub0, sub1)
    wib = (wi0, wi1)
    rowsb = (rows0, rows1)
    wrb = (wr0, wr1)
    semv = (sv0_, sv1_)
    semw = (sw0, sw1)

    def prep(cc, p):
        start = pl.multiple_of(wflat + cc * LOOK, 8)
        pltpu.sync_copy(inp_hbm.at[pl.ds(start, LOOK)], wib[p])
        for m in range(LOOK // 16):
            sl = pl.ds(m * 16, 16)
            offs = lax.rem(io + (m * 16) % FIELDS, FIELDS) * FEAT_PER_FIELD
            full = wib[p][sl] + offs
            wib[p][sl] = full
            idxb[p][sl] = lax.shift_right_logical(full, 4)
            subb[p][sl] = lax.bitwise_and(full, 15)
        for g in range(NGATH):
            isl = idxb[p].at[pl.ds(g * 104, 104)]
            wsl = wib[p].at[pl.ds(g * 104, 104)]
            pltpu.make_async_copy(
                sv_hbm.at[isl], rowsb[p].at[pl.ds(g * 104, 104)],
                semv[p]).start()
            pltpu.make_async_copy(
                w_hbm.at[wsl], wrb[p].at[pl.ds(g * 104, 104)],
                semw[p]).start()

    def compute(cc, p):
        for g in range(NGATH):
            isl = idxb[p].at[pl.ds(g * 104, 104)]
            wsl = wib[p].at[pl.ds(g * 104, 104)]
            pltpu.make_async_copy(
                sv_hbm.at[isl], rowsb[p].at[pl.ds(g * 104, 104)],
                semv[p]).wait()
            pltpu.make_async_copy(
                w_hbm.at[wsl], wrb[p].at[pl.ds(g * 104, 104)],
                semw[p]).wait()
        zero = jnp.zeros((16,), jnp.float32)
        for r in range(ROWS_PER_CHUNK):
            S16 = zero
            Q16 = zero
            for m in range(13):
                d0 = (r * FIELDS + 2 * m) + jhalf
                subg = plsc.load_gather(subb[p], [d0])
                gv = plsc.load_gather(rowsb[p], [d0, subg * 8 + kidx])
                S16 = S16 + gv
                Q16 = Q16 + gv * gv
            Su = S16 + S16.at[permi].get(mode="promise_in_bounds")
            s2dup = jnp.sum(Su * Su)
            qs = jnp.sum(Q16)
            w1i = r * FIELDS + io
            w2i = jnp.where(io < 10, r * FIELDS + 16 + io, 0)
            g1 = plsc.load_gather(wrb[p], [w1i])
            g2 = plsc.load_gather(wrb[p], [w2i], mask=io < 10)
            wsum = jnp.sum(g1 + jnp.where(io < 10, g2, 0.0))
            val = wsum + 0.25 * s2dup - 0.5 * qs
            vid = jnp.full((16,), cc * ROWS_PER_CHUNK + r, jnp.int32)
            plsc.store_scatter(outv, [vid],
                               jnp.full((16,), val, jnp.float32), mask=lane0)

    prep(0, 0)

    def loop_body(i, carry):
        for p in range(2):
            cc = i * 2 + p

            @pl.when(cc + 1 < CHUNKS_B)
            def _():
                prep(cc + 1, (p + 1) % 2)

            compute(cc, p)
        return carry

    lax.fori_loop(0, CHUNKS_B // 2, loop_body, 0)
    obase = pl.multiple_of(wid * ROWS_PER_W, 8)
    pltpu.sync_copy(outv, out_hbm.at[pl.ds(obase, ROWS_PER_W)])


@jax.jit
def _ffm(inputs, w, v):
    mesh = plsc.VectorSubcoreMesh(core_axis_name="c", subcore_axis_name="s")
    params = pltpu.CompilerParams(
        needs_layout_passes=False, use_tc_tiling_on_sc=False)

    vt = jnp.transpose(v, (1, 2, 0))
    sv = pl.pallas_call(
        _svt_block,
        grid=(GRID_A,),
        in_specs=[pl.BlockSpec((FIELDS, KDIM, FBLK), lambda i: (0, 0, i))],
        out_specs=pl.BlockSpec((FBLK // 16, 128), lambda i: (i, 0)),
        out_shape=jax.ShapeDtypeStruct((TOTAL_FEAT // 16, 128), jnp.float32),
    )(vt)

    ffm_fn = pl.kernel(
        _ffm_body,
        mesh=mesh,
        compiler_params=params,
        out_type=jax.ShapeDtypeStruct((BATCH,), jnp.float32),
        scratch_types=[
            pltpu.VMEM((LOOK,), jnp.int32),
            pltpu.VMEM((LOOK,), jnp.int32),
            pltpu.VMEM((LOOK,), jnp.int32),
            pltpu.VMEM((LOOK,), jnp.int32),
            pltpu.VMEM((LOOK,), jnp.int32),
            pltpu.VMEM((LOOK,), jnp.int32),
            pltpu.VMEM((LOOK, 128), jnp.float32),
            pltpu.VMEM((LOOK, 128), jnp.float32),
            pltpu.VMEM((LOOK,), jnp.float32),
            pltpu.VMEM((LOOK,), jnp.float32),
            pltpu.VMEM((ROWS_PER_W,), jnp.float32),
            pltpu.SemaphoreType.DMA,
            pltpu.SemaphoreType.DMA,
            pltpu.SemaphoreType.DMA,
            pltpu.SemaphoreType.DMA,
        ],
    )
    return ffm_fn(inputs.reshape(-1), sv, w.reshape(-1))


def kernel(inputs, w0, w, v):
    out = _ffm(inputs, w, v)
    return out.reshape(BATCH, 1) + w0
